# Initial kernel scaffold; baseline (speedup 1.0000x reference)
import jax
import jax.numpy as jnp
from jax.experimental import pallas as pl


def kernel(invoxel_xyz, invoxel_map, src_feat, voxel_center, W, b):
    raise NotImplementedError("write your pallas kernel here")


# speedup vs baseline: 1.9554x; 1.9554x over previous
import functools

import jax
import jax.numpy as jnp
from jax import lax
from jax.experimental import pallas as pl
from jax.experimental.pallas import tpu as pltpu
from jax.experimental.pallas import tpu_sc as plsc

FUSE_K = 8
D = 128
L = 16
NC, NS = 2, 16
NW = NC * NS
CH = 16
ROWS = CH * FUSE_K
CW = 48


def _prep_body(map_ref, xyz_ref, cen_ref, coef_ref, idx_ref):
    m = map_ref[...]
    mask = m == 0
    gf = m[0:1, :]
    idx_ref[...] = jnp.where(mask, jnp.broadcast_to(gf, m.shape), m)
    xyz = xyz_ref[...]
    cen = cen_ref[...]
    xf = xyz[0:3, :]
    parts = []
    for k in range(FUSE_K):
        xk = xyz[3 * k:3 * k + 3, :]
        mk = jnp.broadcast_to(mask[k:k + 1, :], xk.shape)
        xm = jnp.where(mk, xf, xk)
        off = xm - cen
        dist = jnp.sqrt(jnp.sum(off * off, axis=0, keepdims=True))
        parts.append(xm)
        parts.append(dist)
    parts.append(cen)
    parts.append(jnp.zeros((CW - 3 - 4 * FUSE_K, cen.shape[1]), jnp.float32))
    coef_ref[...] = jnp.concatenate(parts, axis=0)


def _prep(mpT, xyzT, cenT):
    N = mpT.shape[1]
    Bn = 2048
    return pl.pallas_call(
        _prep_body,
        grid=(pl.cdiv(N, Bn),),
        in_specs=[
            pl.BlockSpec((FUSE_K, Bn), lambda i: (0, i)),
            pl.BlockSpec((3 * FUSE_K, Bn), lambda i: (0, i)),
            pl.BlockSpec((3, Bn), lambda i: (0, i)),
        ],
        out_specs=[
            pl.BlockSpec((CW, Bn), lambda i: (0, i)),
            pl.BlockSpec((FUSE_K, Bn), lambda i: (0, i)),
        ],
        out_shape=[
            jax.ShapeDtypeStruct((CW, N), jnp.float32),
            jax.ShapeDtypeStruct((FUSE_K, N), jnp.int32),
        ],
    )(mpT, xyzT, cenT)


_GDN = lax.GatherDimensionNumbers(
    offset_dims=(), collapsed_slice_dims=(0,), start_index_map=(0,))


def _bc(q, lane):
    idx = jnp.full((L, 1), lane, jnp.int32)
    return lax.gather(q, idx, dimension_numbers=_GDN, slice_sizes=(1,),
                      mode=lax.GatherScatterMode.PROMISE_IN_BOUNDS)


def _sc_body(coef_hbm, idx_hbm, feat_hbm, wp_hbm, out_hbm,
             idx_v, rows_v, coef_v, out_v, wp_v, sem):
    cid = lax.axis_index("c")
    sid = lax.axis_index("s")
    wid = sid * NC + cid
    nchunks = coef_hbm.shape[0] // CH
    pltpu.sync_copy(wp_hbm, wp_v)

    def wp(i, c):
        return wp_v[i, pl.ds(L * c, L)]

    def voxel_body(v, _):
        q0 = coef_v[v, pl.ds(0, L)]
        q1 = coef_v[v, pl.ds(L, L)]
        q2 = coef_v[v, pl.ds(2 * L, L)]
        c0 = _bc(q2, 0)
        c1 = _bc(q2, 1)
        c2 = _bc(q2, 2)
        base = [wp(7, c) + c0 * wp(4, c) + c1 * wp(5, c) + c2 * wp(6, c)
                for c in range(D // L)]
        acc = [jnp.zeros((L,), jnp.float32) for _ in range(D // L)]
        for k in range(FUSE_K):
            q = q0 if k < 4 else q1
            o = (k % 4) * 4
            x0 = _bc(q, o)
            x1 = _bc(q, o + 1)
            x2 = _bc(q, o + 2)
            dd = _bc(q, o + 3)
            r = v * FUSE_K + k
            for c in range(D // L):
                row = rows_v[r, pl.ds(L * c, L)]
                w = (base[c] + x0 * wp(0, c) + x1 * wp(1, c)
                     + x2 * wp(2, c) + dd * wp(3, c))
                acc[c] = acc[c] + row * w
        for c in range(D // L):
            out_v[v, pl.ds(L * c, L)] = acc[c]
        return 0

    def chunk_body(t, _):
        chunk = wid + t * NW
        vbase = chunk * CH
        pltpu.sync_copy(idx_hbm.at[chunk], idx_v)
        pltpu.async_copy(feat_hbm.at[idx_v], rows_v, sem).wait()
        pltpu.sync_copy(coef_hbm.at[pl.ds(vbase, CH)], coef_v)
        lax.fori_loop(0, CH, voxel_body, 0, unroll=False)
        pltpu.sync_copy(out_v, out_hbm.at[pl.ds(vbase, CH)])
        return 0

    nch_w = (nchunks - wid + NW - 1) // NW
    lax.fori_loop(0, nch_w, chunk_body, 0, unroll=False)


def _sc_call(coef, idx2d, src_feat, wpack):
    N = coef.shape[0]
    mesh = plsc.VectorSubcoreMesh(core_axis_name="c", subcore_axis_name="s",
                                  num_cores=NC, num_subcores=NS)
    f = pl.kernel(
        _sc_body,
        out_type=jax.ShapeDtypeStruct((N, D), jnp.float32),
        mesh=mesh,
        scratch_types=[
            pltpu.VMEM((ROWS,), jnp.int32),
            pltpu.VMEM((ROWS, D), jnp.float32),
            pltpu.VMEM((CH, CW), jnp.float32),
            pltpu.VMEM((CH, D), jnp.float32),
            pltpu.VMEM((FUSE_K, D), jnp.float32),
            pltpu.SemaphoreType.DMA,
        ],
    )
    return f(coef, idx2d, src_feat, wpack)


def kernel(invoxel_xyz, invoxel_map, src_feat, voxel_center, W, b):
    N = invoxel_map.shape[0]
    xyzT = invoxel_xyz.reshape(N, 3 * FUSE_K).T
    coefT, idxT = _prep(invoxel_map.T, xyzT, voxel_center.T)
    coef = coefT.T
    idx2d = idxT.T.reshape(N * FUSE_K // ROWS, ROWS)
    Wt = W.T
    wpack = jnp.stack([
        Wt[0] + Wt[6], Wt[1] + Wt[7], Wt[2] + Wt[8], Wt[9],
        Wt[3] - Wt[0], Wt[4] - Wt[1], Wt[5] - Wt[2], b,
    ], axis=0) * (1.0 / FUSE_K)
    return _sc_call(coef, idx2d, src_feat, wpack)


# speedup vs baseline: 3.1873x; 1.6300x over previous
import functools

import jax
import jax.numpy as jnp
from jax import lax
from jax.experimental import pallas as pl
from jax.experimental.pallas import tpu as pltpu
from jax.experimental.pallas import tpu_sc as plsc

FUSE_K = 8
D = 128
L = 16
NC, NS = 2, 16
NW = NC * NS
CH = 16
ROWS = CH * FUSE_K
CW = 48


def _prep_body(map_ref, xyz_ref, cen_ref, coef_ref, idx_ref):
    m = map_ref[...]
    mask = m == 0
    gf = m[0:1, :]
    idx_ref[...] = jnp.where(mask, jnp.broadcast_to(gf, m.shape), m)
    xyz = xyz_ref[...]
    cen = cen_ref[...]
    xf = xyz[0:3, :]
    parts = []
    for k in range(FUSE_K):
        xk = xyz[3 * k:3 * k + 3, :]
        mk = jnp.broadcast_to(mask[k:k + 1, :], xk.shape)
        xm = jnp.where(mk, xf, xk)
        off = xm - cen
        dist = jnp.sqrt(jnp.sum(off * off, axis=0, keepdims=True))
        parts.append(xm)
        parts.append(dist)
    parts.append(cen)
    parts.append(jnp.zeros((CW - 3 - 4 * FUSE_K, cen.shape[1]), jnp.float32))
    coef_ref[...] = jnp.concatenate(parts, axis=0)


def _prep(mpT, xyzT, cenT):
    N = mpT.shape[1]
    Bn = 2048
    return pl.pallas_call(
        _prep_body,
        grid=(pl.cdiv(N, Bn),),
        in_specs=[
            pl.BlockSpec((FUSE_K, Bn), lambda i: (0, i)),
            pl.BlockSpec((3 * FUSE_K, Bn), lambda i: (0, i)),
            pl.BlockSpec((3, Bn), lambda i: (0, i)),
        ],
        out_specs=[
            pl.BlockSpec((CW, Bn), lambda i: (0, i)),
            pl.BlockSpec((FUSE_K, Bn), lambda i: (0, i)),
        ],
        out_shape=[
            jax.ShapeDtypeStruct((CW, N), jnp.float32),
            jax.ShapeDtypeStruct((FUSE_K, N), jnp.int32),
        ],
    )(mpT, xyzT, cenT)


_GDN = lax.GatherDimensionNumbers(
    offset_dims=(), collapsed_slice_dims=(0,), start_index_map=(0,))


def _bc(q, lane):
    idx = jnp.full((L, 1), lane, jnp.int32)
    return lax.gather(q, idx, dimension_numbers=_GDN, slice_sizes=(1,),
                      mode=lax.GatherScatterMode.PROMISE_IN_BOUNDS)


def _sc_body(coef_hbm, idx_hbm, feat_hbm, wp_hbm, out_hbm,
             idx_v, rows_v, coef_v, out_v, wp_v,
             sem_g, sem_i, sem_c, sem_o):
    cid = lax.axis_index("c")
    sid = lax.axis_index("s")
    wid = sid * NC + cid
    nchunks = coef_hbm.shape[0] // CH
    nch_w = (nchunks - wid + NW - 1) // NW
    pltpu.sync_copy(wp_hbm, wp_v)

    def wp(i, c):
        return wp_v[i, pl.ds(L * c, L)]

    def chunk_of(t):
        return wid + jnp.minimum(t, nch_w - 1) * NW

    def start_gather(t):
        slot = lax.rem(t, 2)
        pltpu.async_copy(feat_hbm.at[idx_v.at[lax.rem(t, 3)]],
                         rows_v.at[slot], sem_g)

    def start_idx(t):
        pltpu.async_copy(idx_hbm.at[chunk_of(t)], idx_v.at[lax.rem(t, 3)],
                         sem_i)

    def start_coef(t):
        pltpu.async_copy(coef_hbm.at[pl.ds(chunk_of(t) * CH, CH)],
                         coef_v.at[lax.rem(t, 2)], sem_c)

    def wait_g():
        pltpu.make_async_copy(feat_hbm.at[idx_v.at[0]], rows_v.at[0],
                              sem_g).wait()

    def wait_i():
        pltpu.make_async_copy(idx_hbm.at[0], idx_v.at[0], sem_i).wait()

    def wait_c():
        pltpu.make_async_copy(coef_hbm.at[pl.ds(0, CH)], coef_v.at[0],
                              sem_c).wait()

    def wait_o():
        pltpu.make_async_copy(out_v.at[0], out_hbm.at[pl.ds(0, CH)],
                              sem_o).wait()

    def compute(slot, v, _):
        q0 = coef_v[slot, v, pl.ds(0, L)]
        q1 = coef_v[slot, v, pl.ds(L, L)]
        q2 = coef_v[slot, v, pl.ds(2 * L, L)]
        c0 = _bc(q2, 0)
        c1 = _bc(q2, 1)
        c2 = _bc(q2, 2)
        for half in range(2):
            cs = range(half * 4, half * 4 + 4)
            S = [None] * 4
            X0 = [None] * 4
            X1 = [None] * 4
            X2 = [None] * 4
            DD = [None] * 4
            for k in range(FUSE_K):
                q = q0 if k < 4 else q1
                o = (k % 4) * 4
                x0 = _bc(q, o)
                x1 = _bc(q, o + 1)
                x2 = _bc(q, o + 2)
                dd = _bc(q, o + 3)
                r = v * FUSE_K + k
                for i, c in enumerate(cs):
                    row = rows_v[slot, r, pl.ds(L * c, L)]
                    if k == 0:
                        S[i] = row
                        X0[i] = x0 * row
                        X1[i] = x1 * row
                        X2[i] = x2 * row
                        DD[i] = dd * row
                    else:
                        S[i] = S[i] + row
                        X0[i] = X0[i] + x0 * row
                        X1[i] = X1[i] + x1 * row
                        X2[i] = X2[i] + x2 * row
                        DD[i] = DD[i] + dd * row
            for i, c in enumerate(cs):
                base = (wp(7, c) + c0 * wp(4, c) + c1 * wp(5, c)
                        + c2 * wp(6, c))
                res = (base * S[i] + X0[i] * wp(0, c) + X1[i] * wp(1, c)
                       + X2[i] * wp(2, c) + DD[i] * wp(3, c))
                out_v[slot, v, pl.ds(L * c, L)] = res
        return 0

    start_idx(0)
    wait_i()
    start_gather(0)
    start_idx(1)
    start_coef(0)

    def loop_body(t, _):
        slot = lax.rem(t, 2)
        wait_i()
        start_gather(t + 1)
        start_idx(t + 2)
        wait_c()
        start_coef(t + 1)
        wait_g()
        lax.fori_loop(0, CH, functools.partial(compute, slot), 0,
                      unroll=False)

        @pl.when(t > 0)
        def _():
            wait_o()

        pltpu.async_copy(out_v.at[slot],
                         out_hbm.at[pl.ds(chunk_of(t) * CH, CH)], sem_o)
        return 0

    lax.fori_loop(0, nch_w, loop_body, 0, unroll=False)
    wait_g()
    wait_c()
    wait_i()
    wait_o()


def _sc_call(coef, idx2d, src_feat, wpack):
    N = coef.shape[0]
    mesh = plsc.VectorSubcoreMesh(core_axis_name="c", subcore_axis_name="s",
                                  num_cores=NC, num_subcores=NS)
    f = pl.kernel(
        _sc_body,
        out_type=jax.ShapeDtypeStruct((N, D), jnp.float32),
        mesh=mesh,
        scratch_types=[
            pltpu.VMEM((3, ROWS), jnp.int32),
            pltpu.VMEM((2, ROWS, D), jnp.float32),
            pltpu.VMEM((2, CH, CW), jnp.float32),
            pltpu.VMEM((2, CH, D), jnp.float32),
            pltpu.VMEM((FUSE_K, D), jnp.float32),
            pltpu.SemaphoreType.DMA,
            pltpu.SemaphoreType.DMA,
            pltpu.SemaphoreType.DMA,
            pltpu.SemaphoreType.DMA,
        ],
    )
    return f(coef, idx2d, src_feat, wpack)


def kernel(invoxel_xyz, invoxel_map, src_feat, voxel_center, W, b):
    N = invoxel_map.shape[0]
    xyzT = invoxel_xyz.reshape(N, 3 * FUSE_K).T
    coefT, idxT = _prep(invoxel_map.T, xyzT, voxel_center.T)
    coef = coefT.T
    idx2d = idxT.T.reshape(N * FUSE_K // ROWS, ROWS)
    Wt = W.T
    wpack = jnp.stack([
        Wt[0] + Wt[6], Wt[1] + Wt[7], Wt[2] + Wt[8], Wt[9],
        Wt[3] - Wt[0], Wt[4] - Wt[1], Wt[5] - Wt[2], b,
    ], axis=0) * (1.0 / FUSE_K)
    return _sc_call(coef, idx2d, src_feat, wpack)
